# Initial kernel scaffold; baseline (speedup 1.0000x reference)
import jax
import jax.numpy as jnp
from jax.experimental import pallas as pl


def kernel(features, edge_index, fc0_w, fc0_b, fc1_w, fc1_b, conv_w):
    raise NotImplementedError("write your pallas kernel here")


# speedup vs baseline: 6.1631x; 6.1631x over previous
import functools
import math

import jax
import jax.numpy as jnp
from jax import lax
from jax.experimental import pallas as pl
from jax.experimental.pallas import tpu as pltpu
from jax.experimental.pallas import tpu_sc as plsc

N = 10000
E = 320000
IN = 128
HID = 128
OUT = 64
L_LAYERS = 8
ALPHA = 0.1
LAM = 0.5

NC = 2
NS = 16
NW = NC * NS
EPT = E // NW
CH = 80
NCHUNK = EPT // CH
NSLAB = 10
SLAB = N // NSLAB
ZR = 200
DEGW = 128

_mesh = plsc.VectorSubcoreMesh(
    core_axis_name="c", subcore_axis_name="s", num_cores=NC, num_subcores=NS)


def _zero_vmem_rows(buf, nrows, width):
    z16 = jnp.zeros((16,), jnp.float32)

    def body(i, _):
        for k in range(width // 16):
            buf[i, pl.ds(16 * k, 16)] = z16
        return 0

    lax.fori_loop(0, nrows, body, 0, unroll=False)


def _deg_body(dst_hbm, out_hbm, didx, ones_v, zbuf, deg_s):
    cid = lax.axis_index("c")
    sid = lax.axis_index("s")
    wid = sid * NC + cid

    one16 = jnp.ones((16,), jnp.float32)

    def fill_ones(i, _):
        for k in range(DEGW // 16):
            ones_v[i, pl.ds(16 * k, 16)] = one16
        return 0

    lax.fori_loop(0, CH, fill_ones, 0, unroll=False)
    _zero_vmem_rows(zbuf, ZR, DEGW)

    @pl.when(sid < NSLAB)
    def _():
        for j in range(SLAB // ZR):
            pltpu.sync_copy(zbuf, deg_s.at[pl.ds(sid * SLAB + j * ZR, ZR)])
    plsc.subcore_barrier()

    def chunk(j, _):
        b = wid * EPT + j * CH
        pltpu.sync_copy(dst_hbm.at[pl.ds(b, CH)], didx)
        pltpu.sync_copy(ones_v, deg_s.at[didx], add=True)
        return 0

    lax.fori_loop(0, NCHUNK, chunk, 0, unroll=False)
    plsc.subcore_barrier()

    @pl.when(sid < NSLAB)
    def _():
        pltpu.sync_copy(deg_s.at[pl.ds(sid * SLAB, SLAB)],
                        out_hbm.at[cid, pl.ds(sid * SLAB, SLAB)])


_deg_call = pl.kernel(
    _deg_body,
    out_type=jax.ShapeDtypeStruct((NC, N, DEGW), jnp.float32),
    mesh=_mesh,
    scratch_types=[
        pltpu.VMEM((CH,), jnp.int32),
        pltpu.VMEM((CH, DEGW), jnp.float32),
        pltpu.VMEM((ZR, DEGW), jnp.float32),
        pltpu.VMEM_SHARED((N, DEGW), jnp.float32),
    ],
)


def _spmm_body(hs_hbm, src_hbm, dst_hbm, out_hbm,
               sidx, didx, rows, zbuf, agg_s, gsem):
    cid = lax.axis_index("c")
    sid = lax.axis_index("s")
    wid = sid * NC + cid

    _zero_vmem_rows(zbuf, ZR, HID)

    @pl.when(sid < NSLAB)
    def _():
        for j in range(SLAB // ZR):
            pltpu.sync_copy(zbuf, agg_s.at[pl.ds(sid * SLAB + j * ZR, ZR)])
    plsc.subcore_barrier()

    def chunk(j, _):
        b = wid * EPT + j * CH
        pltpu.sync_copy(src_hbm.at[pl.ds(b, CH)], sidx)
        pltpu.sync_copy(dst_hbm.at[pl.ds(b, CH)], didx)
        pltpu.async_copy(hs_hbm.at[sidx], rows, gsem).wait()
        pltpu.sync_copy(rows, agg_s.at[didx], add=True)
        return 0

    lax.fori_loop(0, NCHUNK, chunk, 0, unroll=False)
    plsc.subcore_barrier()

    @pl.when(sid < NSLAB)
    def _():
        pltpu.sync_copy(agg_s.at[pl.ds(sid * SLAB, SLAB)],
                        out_hbm.at[cid, pl.ds(sid * SLAB, SLAB)])


_spmm_call = pl.kernel(
    _spmm_body,
    out_type=jax.ShapeDtypeStruct((NC, N, HID), jnp.float32),
    mesh=_mesh,
    scratch_types=[
        pltpu.VMEM((CH,), jnp.int32),
        pltpu.VMEM((CH,), jnp.int32),
        pltpu.VMEM((CH, HID), jnp.float32),
        pltpu.VMEM((ZR, HID), jnp.float32),
        pltpu.VMEM_SHARED((N, HID), jnp.float32),
        pltpu.SemaphoreType.DMA,
    ],
)


_BLK = 1000
_GRID = N // _BLK


def _proj_body(x_ref, w_ref, b_ref, o_ref):
    o_ref[...] = jnp.maximum(
        jnp.dot(x_ref[...], w_ref[...], preferred_element_type=jnp.float32)
        + b_ref[...], 0.0)


def _proj(x, wt, b):
    return pl.pallas_call(
        _proj_body,
        grid=(_GRID,),
        in_specs=[
            pl.BlockSpec((_BLK, IN), lambda i: (i, 0)),
            pl.BlockSpec((IN, HID), lambda i: (0, 0)),
            pl.BlockSpec((1, HID), lambda i: (0, 0)),
        ],
        out_specs=pl.BlockSpec((_BLK, HID), lambda i: (i, 0)),
        out_shape=jax.ShapeDtypeStruct((N, HID), jnp.float32),
    )(x, wt, b)


def _norm_body(d0_ref, d1_ref, h_ref, n_ref, hs_ref):
    deg = d0_ref[:, 0:1] + d1_ref[:, 0:1]
    nrm = lax.rsqrt(jnp.maximum(deg, 1.0))
    n_ref[...] = nrm
    hs_ref[...] = h_ref[...] * nrm


def _norm(d0, d1, h):
    return pl.pallas_call(
        _norm_body,
        grid=(_GRID,),
        in_specs=[
            pl.BlockSpec((_BLK, DEGW), lambda i: (i, 0)),
            pl.BlockSpec((_BLK, DEGW), lambda i: (i, 0)),
            pl.BlockSpec((_BLK, HID), lambda i: (i, 0)),
        ],
        out_specs=[
            pl.BlockSpec((_BLK, 1), lambda i: (i, 0)),
            pl.BlockSpec((_BLK, HID), lambda i: (i, 0)),
        ],
        out_shape=[
            jax.ShapeDtypeStruct((N, 1), jnp.float32),
            jax.ShapeDtypeStruct((N, HID), jnp.float32),
        ],
    )(d0, d1, h)


def _combine_body(beta, a0_ref, a1_ref, h0_ref, n_ref, w_ref, h_ref, hs_ref):
    agg = (a0_ref[...] + a1_ref[...]) * n_ref[...]
    r = (1.0 - ALPHA) * agg + ALPHA * h0_ref[...]
    r = (1.0 - beta) * r + beta * jnp.dot(
        r, w_ref[...], preferred_element_type=jnp.float32)
    h = jnp.maximum(r, 0.0)
    h_ref[...] = h
    hs_ref[...] = h * n_ref[...]


def _combine(a0, a1, h0, nrm, w, beta):
    return pl.pallas_call(
        functools.partial(_combine_body, beta),
        grid=(_GRID,),
        in_specs=[
            pl.BlockSpec((_BLK, HID), lambda i: (i, 0)),
            pl.BlockSpec((_BLK, HID), lambda i: (i, 0)),
            pl.BlockSpec((_BLK, HID), lambda i: (i, 0)),
            pl.BlockSpec((_BLK, 1), lambda i: (i, 0)),
            pl.BlockSpec((HID, HID), lambda i: (0, 0)),
        ],
        out_specs=[
            pl.BlockSpec((_BLK, HID), lambda i: (i, 0)),
            pl.BlockSpec((_BLK, HID), lambda i: (i, 0)),
        ],
        out_shape=[
            jax.ShapeDtypeStruct((N, HID), jnp.float32),
            jax.ShapeDtypeStruct((N, HID), jnp.float32),
        ],
    )(a0, a1, h0, nrm, w)


def _out_body(h_ref, w_ref, b_ref, o_ref):
    o = jnp.dot(h_ref[...], w_ref[...],
                preferred_element_type=jnp.float32) + b_ref[...]
    m = jnp.max(o, axis=1, keepdims=True)
    e = jnp.exp(o - m)
    lse = jnp.log(jnp.sum(e, axis=1, keepdims=True)) + m
    o_ref[...] = o - lse


def _outproj(h, wt, b):
    return pl.pallas_call(
        _out_body,
        grid=(_GRID,),
        in_specs=[
            pl.BlockSpec((_BLK, HID), lambda i: (i, 0)),
            pl.BlockSpec((HID, OUT), lambda i: (0, 0)),
            pl.BlockSpec((1, OUT), lambda i: (0, 0)),
        ],
        out_specs=pl.BlockSpec((_BLK, OUT), lambda i: (i, 0)),
        out_shape=jax.ShapeDtypeStruct((N, OUT), jnp.float32),
    )(h, wt, b)


def kernel(features, edge_index, fc0_w, fc0_b, fc1_w, fc1_b, conv_w):
    src = edge_index[0]
    dst = edge_index[1]

    deg2 = _deg_call(dst)
    h0 = _proj(features, fc0_w.T, fc0_b.reshape(1, HID))
    nrm, hs = _norm(deg2[0], deg2[1], h0)

    h = h0
    for i in range(L_LAYERS):
        agg2 = _spmm_call(hs, src, dst)
        beta = math.log(LAM / (i + 1) + 1.0)
        h, hs = _combine(agg2[0], agg2[1], h0, nrm, conv_w[i], beta)

    return _outproj(h, fc1_w.T, fc1_b.reshape(1, OUT))
